# baseline (speedup 1.0000x reference)
import jax
import jax.numpy as jnp
from jax.experimental import pallas as pl

B, T, D = 4096, 10, 900
NA = 16
NB = 16
NW = NA + NB

BS = 128


def _fused_kernel(x_ref, w_ref, bias_ref, af_ref, ap_ref, fwd_ref, back_ref):
    w = w_ref[...]
    bias = bias_ref[...]
    col = jax.lax.broadcasted_iota(jnp.int32, (BS, NW), 1)
    is_f = col < NA
    for t in range(T):
        x = x_ref[:, t, :]
        logits = jnp.dot(x, w, preferred_element_type=jnp.float32) + bias
        m = jnp.max(logits, axis=1, keepdims=True)
        e = jnp.exp(logits - m)
        af = af_ref[:, t:t + 1]
        ap = ap_ref[:, t:t + 1]
        num_f = jnp.sum(jnp.where(col == af, e, 0.0), axis=1, keepdims=True)
        num_b = jnp.sum(jnp.where(col == ap + NA, e, 0.0), axis=1, keepdims=True)
        den_f = jnp.sum(jnp.where(is_f, e, 0.0), axis=1, keepdims=True)
        den_b = jnp.sum(jnp.where(is_f, 0.0, e), axis=1, keepdims=True)
        fwd_ref[:, t:t + 1] = jnp.where(af == -1, 1.0, num_f / den_f)
        back_ref[:, t:t + 1] = jnp.where((ap == -1) | (ap == 2), 1.0,
                                         num_b / den_b)


def kernel(traj, actions, Wf, bf, Wb, bb, Wr, br):
    w = jnp.concatenate([Wf, Wb], axis=1)
    bias = jnp.concatenate([bf, bb]).reshape(1, NW)
    acts = actions.astype(jnp.int32)
    acts_prev = jnp.concatenate(
        [jnp.full((B, 1), -3, jnp.int32), acts[:, :-1]], axis=1)

    grid = (B // BS,)
    fwd_sel, back_full = pl.pallas_call(
        _fused_kernel,
        grid=grid,
        in_specs=[
            pl.BlockSpec((BS, T, D), lambda i: (i, 0, 0)),
            pl.BlockSpec((D, NW), lambda i: (0, 0)),
            pl.BlockSpec((1, NW), lambda i: (0, 0)),
            pl.BlockSpec((BS, T), lambda i: (i, 0)),
            pl.BlockSpec((BS, T), lambda i: (i, 0)),
        ],
        out_specs=[
            pl.BlockSpec((BS, T), lambda i: (i, 0)),
            pl.BlockSpec((BS, T), lambda i: (i, 0)),
        ],
        out_shape=[
            jax.ShapeDtypeStruct((B, T), jnp.float32),
            jax.ShapeDtypeStruct((B, T), jnp.float32),
        ],
    )(traj, w, bias, acts, acts_prev)

    rewards = jnp.zeros((0, 1), dtype=jnp.float32)
    return (fwd_sel, back_full[:, 1:], rewards)


# speedup vs baseline: 1.0483x; 1.0483x over previous
import functools

import jax
import jax.numpy as jnp
from jax import lax
from jax.experimental import pallas as pl
from jax.experimental.pallas import tpu as pltpu
from jax.experimental.pallas import tpu_sc as plsc

B, T, D = 4096, 10, 900
NA = 16
NB = 16
NW = NA + NB

BS = 256

_NC = 2
_NS = 16
_L = 16
_NTILES = _NC * _NS
_SPT = B // _NTILES
_HTILES = _NTILES // 2
_NG = _SPT // _L


def _matmul_kernel(x_ref, w_ref, bias_ref, lt_ref):
    w = w_ref[...]
    bias = bias_ref[...]
    for t in range(T):
        x = x_ref[:, t, :]
        lt = lax.dot_general(w, x, (((0,), (1,)), ((), ())),
                             preferred_element_type=jnp.float32)
        lt_ref[t * NW:(t + 1) * NW, :] = lt + bias


def _sc_sample(lt_hbm, aft_hbm, fwd_hbm, back_hbm, lt_v, af_v, fwd_v, back_v):
    wid = lax.axis_index("s") * _NC + lax.axis_index("c")
    base = wid * _SPT

    @pl.when(wid < _HTILES)
    def _():
        _sc_sample_body(lt_hbm, aft_hbm, fwd_hbm, back_hbm,
                        lt_v, af_v, fwd_v, back_v, base)


def _sc_sample_body(lt_hbm, aft_hbm, fwd_hbm, back_hbm,
                    lt_v, af_v, fwd_v, back_v, base):
    pltpu.sync_copy(lt_hbm.at[:, pl.ds(base, _SPT)], lt_v)
    pltpu.sync_copy(aft_hbm.at[:, pl.ds(base, _SPT)], af_v)

    def step(t, carry):
        tp = jnp.maximum(t - 1, 0)
        for g in range(_NG):
            sl = pl.ds(g * _L, _L)
            af = af_v[t, sl]
            ap = af_v[tp, sl]
            logits = [lt_v[t * NW + k, sl] for k in range(NW)]
            m = logits[0]
            for k in range(1, NW):
                m = jnp.maximum(m, logits[k])
            den_f = jnp.zeros((_L,), jnp.float32)
            den_b = jnp.zeros((_L,), jnp.float32)
            num_f = jnp.zeros((_L,), jnp.float32)
            num_b = jnp.zeros((_L,), jnp.float32)
            for k in range(NA):
                e = jnp.exp(logits[k] - m)
                den_f = den_f + e
                num_f = num_f + jnp.where(af == k, e, 0.0)
            for k in range(NB):
                e = jnp.exp(logits[NA + k] - m)
                den_b = den_b + e
                num_b = num_b + jnp.where(ap == k, e, 0.0)
            fwd = jnp.where(af == -1, 1.0, num_f / den_f)
            back = jnp.where((ap == -1) | (ap == 2), 1.0, num_b / den_b)
            fwd_v[t, sl] = fwd
            back_v[t, sl] = back
        return carry

    lax.fori_loop(0, T, step, 0)
    pltpu.sync_copy(fwd_v, fwd_hbm.at[:, pl.ds(base, _SPT)])
    pltpu.sync_copy(back_v, back_hbm.at[:, pl.ds(base, _SPT)])


def kernel(traj, actions, Wf, bf, Wb, bb, Wr, br):
    w = jnp.concatenate([Wf, Wb], axis=1)
    bias = jnp.concatenate([bf, bb]).reshape(NW, 1)
    aft = actions.astype(jnp.int32).T

    H = B // 2
    hgrid = (H // BS,)

    def tc_half(off):
        return pl.pallas_call(
            _matmul_kernel,
            grid=hgrid,
            in_specs=[
                pl.BlockSpec((BS, T, D), lambda i: (off + i, 0, 0)),
                pl.BlockSpec((D, NW), lambda i: (0, 0)),
                pl.BlockSpec((NW, 1), lambda i: (0, 0)),
            ],
            out_specs=pl.BlockSpec((T * NW, BS), lambda i: (0, i)),
            out_shape=jax.ShapeDtypeStruct((T * NW, H), jnp.float32),
        )(traj, w, bias)

    sampler = functools.partial(
        pl.kernel,
        mesh=plsc.VectorSubcoreMesh(core_axis_name="c", subcore_axis_name="s",
                                    num_cores=_NC, num_subcores=_NS),
        out_type=[jax.ShapeDtypeStruct((T, H), jnp.float32),
                  jax.ShapeDtypeStruct((T, H), jnp.float32)],
        scratch_types=[
            pltpu.VMEM((T * NW, _SPT), jnp.float32),
            pltpu.VMEM((T, _SPT), jnp.int32),
            pltpu.VMEM((T, _SPT), jnp.float32),
            pltpu.VMEM((T, _SPT), jnp.float32),
        ],
    )(_sc_sample)

    lt1 = tc_half(0)
    lt2 = tc_half(H // BS)
    fwd1, back1 = sampler(lt1, aft[:, :H])
    fwd2, back2 = sampler(lt2, aft[:, H:])
    fwd_t = jnp.concatenate([fwd1, fwd2], axis=1)
    back_t = jnp.concatenate([back1, back2], axis=1)

    rewards = jnp.zeros((0, 1), dtype=jnp.float32)
    return (fwd_t.T, back_t.T[:, 1:], rewards)


# speedup vs baseline: 1.0676x; 1.0185x over previous
import functools

import jax
import jax.numpy as jnp
from jax import lax
from jax.experimental import pallas as pl
from jax.experimental.pallas import tpu as pltpu
from jax.experimental.pallas import tpu_sc as plsc

B, T, D = 4096, 10, 900
NA = 16
NB = 16
NW = NA + NB

BS = 256

_NC = 2
_NS = 16
_L = 16
_NTILES = _NC * _NS
_SPT = B // _NTILES
_NG = _SPT // _L


def _matmul_kernel(x_ref, w_ref, bias_ref, lt_ref):
    w = w_ref[...]
    bias = bias_ref[...]
    for t in range(T):
        x = x_ref[:, t, :]
        lt = lax.dot_general(w, x, (((0,), (1,)), ((), ())),
                             preferred_element_type=jnp.float32)
        lt_ref[t * NW:(t + 1) * NW, :] = lt + bias


def _sc_sample(lt_hbm, aft_hbm, fwd_hbm, back_hbm, lt_v, af_v, fwd_v, back_v):
    wid = lax.axis_index("s") * _NC + lax.axis_index("c")
    base = wid * _SPT
    pltpu.sync_copy(lt_hbm.at[:, pl.ds(base, _SPT)], lt_v)
    pltpu.sync_copy(aft_hbm.at[:, pl.ds(base, _SPT)], af_v)

    def step(t, carry):
        tp = jnp.maximum(t - 1, 0)
        for g in range(_NG):
            sl = pl.ds(g * _L, _L)
            af = af_v[t, sl]
            ap = af_v[tp, sl]
            logits = [lt_v[t * NW + k, sl] for k in range(NW)]
            m = logits[0]
            for k in range(1, NW):
                m = jnp.maximum(m, logits[k])
            den_f = jnp.zeros((_L,), jnp.float32)
            den_b = jnp.zeros((_L,), jnp.float32)
            num_f = jnp.zeros((_L,), jnp.float32)
            num_b = jnp.zeros((_L,), jnp.float32)
            for k in range(NA):
                e = jnp.exp(logits[k] - m)
                den_f = den_f + e
                num_f = num_f + jnp.where(af == k, e, 0.0)
            for k in range(NB):
                e = jnp.exp(logits[NA + k] - m)
                den_b = den_b + e
                num_b = num_b + jnp.where(ap == k, e, 0.0)
            fwd = jnp.where(af == -1, 1.0, num_f / den_f)
            back = jnp.where((ap == -1) | (ap == 2), 1.0, num_b / den_b)
            fwd_v[t, sl] = fwd
            back_v[t, sl] = back
        return carry

    lax.fori_loop(0, T, step, 0)
    pltpu.sync_copy(fwd_v, fwd_hbm.at[:, pl.ds(base, _SPT)])
    pltpu.sync_copy(back_v, back_hbm.at[:, pl.ds(base, _SPT)])


def kernel(traj, actions, Wf, bf, Wb, bb, Wr, br):
    w = jnp.concatenate([Wf, Wb], axis=1)
    bias = jnp.concatenate([bf, bb]).reshape(NW, 1)
    aft = actions.astype(jnp.int32).T

    grid = (B // BS,)
    lt = pl.pallas_call(
        _matmul_kernel,
        grid=grid,
        in_specs=[
            pl.BlockSpec((BS, T, D), lambda i: (i, 0, 0)),
            pl.BlockSpec((D, NW), lambda i: (0, 0)),
            pl.BlockSpec((NW, 1), lambda i: (0, 0)),
        ],
        out_specs=pl.BlockSpec((T * NW, BS), lambda i: (0, i)),
        out_shape=jax.ShapeDtypeStruct((T * NW, B), jnp.float32),
    )(traj, w, bias)

    sampler = functools.partial(
        pl.kernel,
        mesh=plsc.VectorSubcoreMesh(core_axis_name="c", subcore_axis_name="s",
                                    num_cores=_NC, num_subcores=_NS),
        out_type=[jax.ShapeDtypeStruct((T, B), jnp.float32),
                  jax.ShapeDtypeStruct((T, B), jnp.float32)],
        scratch_types=[
            pltpu.VMEM((T * NW, _SPT), jnp.float32),
            pltpu.VMEM((T, _SPT), jnp.int32),
            pltpu.VMEM((T, _SPT), jnp.float32),
            pltpu.VMEM((T, _SPT), jnp.float32),
        ],
    )(_sc_sample)
    fwd_t, back_t = sampler(lt, aft)

    rewards = jnp.zeros((0, 1), dtype=jnp.float32)
    return (fwd_t.T, back_t.T[:, 1:], rewards)
